# Initial kernel scaffold; baseline (speedup 1.0000x reference)
import jax
import jax.numpy as jnp
from jax.experimental import pallas as pl


def kernel(feat, params):
    raise NotImplementedError("write your pallas kernel here")


# speedup vs baseline: 6.5282x; 6.5282x over previous
import functools
import jax
import jax.numpy as jnp
from jax import lax
from jax.experimental import pallas as pl
from jax.experimental.pallas import tpu as pltpu
from jax.experimental.pallas import tpu_sc as plsc

K = 10
KPAD = 16


def _pre_body(x_ref, mean_ref, s_ref, gamma_ref, beta_ref, w_ref, b_ref, o_ref):
    x = x_ref[...]
    xn = (x - mean_ref[...]) / s_ref[...] * gamma_ref[...] + beta_ref[...]
    o_ref[...] = jax.nn.relu(jnp.dot(xn, w_ref[...]) + b_ref[...])


def _pre_call(x2d, mean, s, gamma, beta, w, b):
    M, C = x2d.shape
    CO = w.shape[1]
    R = 512
    vec = lambda: pl.BlockSpec((1, C), lambda i: (0, 0))
    return pl.pallas_call(
        _pre_body,
        grid=(M // R,),
        in_specs=[
            pl.BlockSpec((R, C), lambda i: (i, 0)),
            vec(), vec(), vec(), vec(),
            pl.BlockSpec((C, CO), lambda i: (0, 0)),
            pl.BlockSpec((1, CO), lambda i: (0, 0)),
        ],
        out_specs=pl.BlockSpec((R, CO), lambda i: (i, 0)),
        out_shape=jax.ShapeDtypeStruct((M, CO), jnp.float32),
    )(x2d, mean, s, gamma, beta, w, b)


def _knn_body(xr_ref, xc_ref, sqr_ref, sqc_ref, idx_ref, *, n_total):
    b = pl.program_id(0)
    xr = xr_ref[0]
    xc = xc_ref[0]
    sq_r = sqr_ref[0, 0]
    sq_c = sqc_ref[0, 0]
    inner = lax.dot_general(xr, xc, (((1,), (1,)), ((), ())),
                            preferred_element_type=jnp.float32)
    dist = sq_r[:, None] - 2.0 * inner + sq_c[None, :]
    cols = lax.broadcasted_iota(jnp.int32, dist.shape, 1)
    base = b * n_total
    for j in range(K):
        m = jnp.min(dist, axis=1)
        idxj = jnp.min(jnp.where(dist == m[:, None], cols, n_total), axis=1)
        idx_ref[0, j, :] = idxj + base
        dist = jnp.where(cols == idxj[:, None], jnp.float32(jnp.inf), dist)
    zero = jnp.zeros_like(cols[:, 0])
    for j in range(K, KPAD):
        idx_ref[0, j, :] = zero


def _knn_call(x, sq):
    B, N, C = x.shape
    R = 256
    return pl.pallas_call(
        functools.partial(_knn_body, n_total=N),
        grid=(B, N // R),
        in_specs=[
            pl.BlockSpec((1, R, C), lambda b, r: (b, r, 0)),
            pl.BlockSpec((1, N, C), lambda b, r: (b, 0, 0)),
            pl.BlockSpec((1, 1, R), lambda b, r: (b, 0, r)),
            pl.BlockSpec((1, 1, N), lambda b, r: (b, 0, 0)),
        ],
        out_specs=pl.BlockSpec((1, KPAD, R), lambda b, r: (b, 0, r)),
        out_shape=jax.ShapeDtypeStruct((B, KPAD, N), jnp.int32),
    )(x, x, sq, sq)


def _gather_call(table, idx3):
    V, C = table.shape
    NW, NCHUNK, CH = idx3.shape
    E = NW * NCHUNK * CH
    info = plsc.get_sparse_core_info()
    assert NW == info.num_cores * info.num_subcores

    mesh = plsc.VectorSubcoreMesh(core_axis_name="c", subcore_axis_name="s")

    @functools.partial(
        pl.kernel,
        mesh=mesh,
        out_type=jax.ShapeDtypeStruct((E, C), jnp.float32),
        scratch_types=[
            pltpu.VMEM((NCHUNK, CH), jnp.int32),
            pltpu.VMEM((CH, C), jnp.float32),
            pltpu.SemaphoreType.DMA,
        ],
    )
    def gk(table_hbm, idx_hbm, out_hbm, idx_v, rows_v, sem):
        wid = lax.axis_index("s") * info.num_cores + lax.axis_index("c")
        pltpu.sync_copy(idx_hbm.at[wid], idx_v)
        base = wid * (NCHUNK * CH)

        def body(ci, carry):
            pltpu.async_copy(table_hbm.at[idx_v.at[ci]], rows_v, sem).wait()
            pltpu.sync_copy(rows_v, out_hbm.at[pl.ds(base + ci * CH, CH)])
            return carry

        lax.fori_loop(0, NCHUNK, body, 0)

    return gk(table, idx3)


def _edge_body(x_ref, g_ref, w1_ref, b1_ref, w2_ref, b2_ref, w3_ref, b3_ref,
               o_ref):
    xc = x_ref[0]
    acc = None
    for j in range(K):
        nb = g_ref[0, j]
        h = jnp.concatenate([xc, nb - xc], axis=-1)
        h = jax.nn.relu(jnp.dot(h, w1_ref[...]) + b1_ref[...])
        h = jax.nn.relu(jnp.dot(h, w2_ref[...]) + b2_ref[...])
        h = jax.nn.relu(jnp.dot(h, w3_ref[...]) + b3_ref[...])
        acc = h if acc is None else jnp.maximum(acc, h)
    o_ref[0] = acc


def _edge_call(x, G, layers):
    B, N, C = x.shape
    R = 512
    CO = layers[2]['W'].shape[1]
    w1, w2, w3 = layers[0]['W'], layers[1]['W'], layers[2]['W']
    b1 = layers[0]['b'].reshape(1, -1)
    b2 = layers[1]['b'].reshape(1, -1)
    b3 = layers[2]['b'].reshape(1, -1)
    wspec = lambda w: pl.BlockSpec(w.shape, lambda b, r: (0, 0))
    return pl.pallas_call(
        _edge_body,
        grid=(B, N // R),
        in_specs=[
            pl.BlockSpec((1, R, C), lambda b, r: (b, r, 0)),
            pl.BlockSpec((1, K, R, C), lambda b, r: (b, 0, r, 0)),
            wspec(w1), wspec(b1), wspec(w2), wspec(b2), wspec(w3), wspec(b3),
        ],
        out_specs=pl.BlockSpec((1, R, CO), lambda b, r: (b, r, 0)),
        out_shape=jax.ShapeDtypeStruct((B, N, CO), jnp.float32),
    )(x, G, w1, b1, w2, b2, w3, b3)


def _final_body(f1_ref, f2_ref, f3_ref, w1_ref, b1_ref, w2_ref, b2_ref, o_ref):
    h = jnp.concatenate([f1_ref[0], f2_ref[0], f3_ref[0]], axis=-1)
    h = jax.nn.relu(jnp.dot(h, w1_ref[...]) + b1_ref[...])
    o_ref[0] = jax.nn.relu(jnp.dot(h, w2_ref[...]) + b2_ref[...])


def _final_call(f1, f2, f3, mlp2):
    B, N, C = f1.shape
    R = 512
    w1, w2 = mlp2[0]['W'], mlp2[1]['W']
    b1 = mlp2[0]['b'].reshape(1, -1)
    b2 = mlp2[1]['b'].reshape(1, -1)
    CO = w2.shape[1]
    fspec = pl.BlockSpec((1, R, C), lambda b, r: (b, r, 0))
    wspec = lambda w: pl.BlockSpec(w.shape, lambda b, r: (0, 0))
    return pl.pallas_call(
        _final_body,
        grid=(B, N // R),
        in_specs=[fspec, fspec, fspec,
                  wspec(w1), wspec(b1), wspec(w2), wspec(b2)],
        out_specs=pl.BlockSpec((1, R, CO), lambda b, r: (b, r, 0)),
        out_shape=jax.ShapeDtypeStruct((B, N, CO), jnp.float32),
    )(f1, f2, f3, w1, b1, w2, b2)


def _edge_layer(x, layers):
    B, N, C = x.shape
    sq = jnp.sum(x * x, axis=-1).reshape(B, 1, N)
    idx = _knn_call(x, sq)
    idx3 = idx[:, :K, :].reshape(32, -1, 128)
    xpad = jnp.concatenate(
        [x.reshape(B * N, C), jnp.zeros((B * N, 128 - C), jnp.float32)], axis=1)
    g = _gather_call(xpad, idx3)
    G = g[:, :C].reshape(B, K, N, C)
    return _edge_call(x, G, layers)


def kernel(feat, params):
    B, N, C0 = feat.shape
    bn = params['bn']
    mean = bn['mean'].reshape(1, -1)
    s = jnp.sqrt(bn['var'] + 1e-5).reshape(1, -1)
    gamma = bn['gamma'].reshape(1, -1)
    beta = bn['beta'].reshape(1, -1)
    x0 = _pre_call(feat.reshape(B * N, C0), mean, s, gamma, beta,
                   params['pre']['W'], params['pre']['b'].reshape(1, -1))
    x0 = x0.reshape(B, N, -1)
    x1 = _edge_layer(x0, params['ec'][0])
    f1 = _edge_layer(x1, params['ec'][1])
    f2 = _edge_layer(f1, params['ec'][2])
    f3 = _edge_layer(f2, params['ec'][3])
    return _final_call(f1, f2, f3, params['mlp2'])


# speedup vs baseline: 9.0876x; 1.3920x over previous
import functools
import jax
import jax.numpy as jnp
from jax import lax
from jax.experimental import pallas as pl
from jax.experimental.pallas import tpu as pltpu
from jax.experimental.pallas import tpu_sc as plsc

K = 10
KPAD = 16


def _pre_body(x_ref, mean_ref, s_ref, gamma_ref, beta_ref, w_ref, b_ref, o_ref):
    x = x_ref[...]
    xn = (x - mean_ref[...]) / s_ref[...] * gamma_ref[...] + beta_ref[...]
    o_ref[...] = jax.nn.relu(jnp.dot(xn, w_ref[...]) + b_ref[...])


def _pre_call(x2d, mean, s, gamma, beta, w, b):
    M, C = x2d.shape
    CO = w.shape[1]
    R = 512
    vec = lambda: pl.BlockSpec((1, C), lambda i: (0, 0))
    return pl.pallas_call(
        _pre_body,
        grid=(M // R,),
        in_specs=[
            pl.BlockSpec((R, C), lambda i: (i, 0)),
            vec(), vec(), vec(), vec(),
            pl.BlockSpec((C, CO), lambda i: (0, 0)),
            pl.BlockSpec((1, CO), lambda i: (0, 0)),
        ],
        out_specs=pl.BlockSpec((R, CO), lambda i: (i, 0)),
        out_shape=jax.ShapeDtypeStruct((M, CO), jnp.float32),
    )(x2d, mean, s, gamma, beta, w, b)


def _knn_body(xr_ref, xc_ref, sqr_ref, sqc_ref, idx_ref, *, n_total):
    b = pl.program_id(0)
    xr = xr_ref[0]
    xc = xc_ref[0]
    sq_r = sqr_ref[0, 0]
    sq_c = sqc_ref[0, 0]
    inner = lax.dot_general(xr, xc, (((1,), (1,)), ((), ())),
                            preferred_element_type=jnp.float32)
    dist = sq_r[:, None] - 2.0 * inner + sq_c[None, :]
    cols = lax.broadcasted_iota(jnp.int32, dist.shape, 1)
    base = b * n_total
    for j in range(K):
        m = jnp.min(dist, axis=1)
        idxj = jnp.min(jnp.where(dist == m[:, None], cols, n_total), axis=1)
        idx_ref[0, j, :] = idxj + base
        dist = jnp.where(cols == idxj[:, None], jnp.float32(jnp.inf), dist)
    zero = jnp.zeros_like(cols[:, 0])
    for j in range(K, KPAD):
        idx_ref[0, j, :] = zero


def _knn_call(x, sq):
    B, N, C = x.shape
    R = 256
    return pl.pallas_call(
        functools.partial(_knn_body, n_total=N),
        grid=(B, N // R),
        in_specs=[
            pl.BlockSpec((1, R, C), lambda b, r: (b, r, 0)),
            pl.BlockSpec((1, N, C), lambda b, r: (b, 0, 0)),
            pl.BlockSpec((1, 1, R), lambda b, r: (b, 0, r)),
            pl.BlockSpec((1, 1, N), lambda b, r: (b, 0, 0)),
        ],
        out_specs=pl.BlockSpec((1, KPAD, R), lambda b, r: (b, 0, r)),
        out_shape=jax.ShapeDtypeStruct((B, KPAD, N), jnp.int32),
    )(x, x, sq, sq)


def _gather_call(table, idx3):
    V, C = table.shape
    NW, NCHUNK, CH = idx3.shape
    E = NW * NCHUNK * CH
    info = plsc.get_sparse_core_info()
    assert NW == info.num_cores * info.num_subcores

    mesh = plsc.VectorSubcoreMesh(core_axis_name="c", subcore_axis_name="s")

    @functools.partial(
        pl.kernel,
        mesh=mesh,
        out_type=jax.ShapeDtypeStruct((E, C), jnp.float32),
        scratch_types=[
            pltpu.VMEM_SHARED((V, C), jnp.float32),
            pltpu.VMEM((NCHUNK, CH), jnp.int32),
            pltpu.VMEM((CH, C), jnp.float32),
            pltpu.SemaphoreType.DMA,
        ],
    )
    def gk(table_hbm, idx_hbm, out_hbm, table_sp, idx_v, rows_v, sem):
        sid = lax.axis_index("s")
        cid = lax.axis_index("c")
        wid = sid * info.num_cores + cid

        @pl.when(sid == 0)
        def _():
            pltpu.sync_copy(table_hbm, table_sp)

        pltpu.sync_copy(idx_hbm.at[wid], idx_v)
        plsc.subcore_barrier()
        base = wid * (NCHUNK * CH)

        def body(ci, carry):
            pltpu.async_copy(table_sp.at[idx_v.at[ci]], rows_v, sem).wait()
            pltpu.sync_copy(rows_v, out_hbm.at[pl.ds(base + ci * CH, CH)])
            return carry

        lax.fori_loop(0, NCHUNK, body, 0)

    return gk(table, idx3)


def _edge_body(x_ref, g_ref, w1_ref, b1_ref, w2_ref, b2_ref, w3_ref, b3_ref,
               o_ref):
    xc = x_ref[0]
    acc = None
    for j in range(K):
        nb = g_ref[0, j]
        h = jnp.concatenate([xc, nb - xc], axis=-1)
        h = jax.nn.relu(jnp.dot(h, w1_ref[...]) + b1_ref[...])
        h = jax.nn.relu(jnp.dot(h, w2_ref[...]) + b2_ref[...])
        h = jax.nn.relu(jnp.dot(h, w3_ref[...]) + b3_ref[...])
        acc = h if acc is None else jnp.maximum(acc, h)
    o_ref[0] = acc


def _edge_call(x, G, layers):
    B, N, C = x.shape
    R = 512
    CO = layers[2]['W'].shape[1]
    w1, w2, w3 = layers[0]['W'], layers[1]['W'], layers[2]['W']
    b1 = layers[0]['b'].reshape(1, -1)
    b2 = layers[1]['b'].reshape(1, -1)
    b3 = layers[2]['b'].reshape(1, -1)
    wspec = lambda w: pl.BlockSpec(w.shape, lambda b, r: (0, 0))
    return pl.pallas_call(
        _edge_body,
        grid=(B, N // R),
        in_specs=[
            pl.BlockSpec((1, R, C), lambda b, r: (b, r, 0)),
            pl.BlockSpec((1, K, R, C), lambda b, r: (b, 0, r, 0)),
            wspec(w1), wspec(b1), wspec(w2), wspec(b2), wspec(w3), wspec(b3),
        ],
        out_specs=pl.BlockSpec((1, R, CO), lambda b, r: (b, r, 0)),
        out_shape=jax.ShapeDtypeStruct((B, N, CO), jnp.float32),
    )(x, G, w1, b1, w2, b2, w3, b3)


def _final_body(f1_ref, f2_ref, f3_ref, w1_ref, b1_ref, w2_ref, b2_ref, o_ref):
    h = jnp.concatenate([f1_ref[0], f2_ref[0], f3_ref[0]], axis=-1)
    h = jax.nn.relu(jnp.dot(h, w1_ref[...]) + b1_ref[...])
    o_ref[0] = jax.nn.relu(jnp.dot(h, w2_ref[...]) + b2_ref[...])


def _final_call(f1, f2, f3, mlp2):
    B, N, C = f1.shape
    R = 512
    w1, w2 = mlp2[0]['W'], mlp2[1]['W']
    b1 = mlp2[0]['b'].reshape(1, -1)
    b2 = mlp2[1]['b'].reshape(1, -1)
    CO = w2.shape[1]
    fspec = pl.BlockSpec((1, R, C), lambda b, r: (b, r, 0))
    wspec = lambda w: pl.BlockSpec(w.shape, lambda b, r: (0, 0))
    return pl.pallas_call(
        _final_body,
        grid=(B, N // R),
        in_specs=[fspec, fspec, fspec,
                  wspec(w1), wspec(b1), wspec(w2), wspec(b2)],
        out_specs=pl.BlockSpec((1, R, CO), lambda b, r: (b, r, 0)),
        out_shape=jax.ShapeDtypeStruct((B, N, CO), jnp.float32),
    )(f1, f2, f3, w1, b1, w2, b2)


def _edge_layer(x, layers):
    B, N, C = x.shape
    sq = jnp.sum(x * x, axis=-1).reshape(B, 1, N)
    idx = _knn_call(x, sq)
    idx3 = idx[:, :K, :].reshape(32, -1, 128)
    xpad = jnp.concatenate(
        [x.reshape(B * N, C), jnp.zeros((B * N, 128 - C), jnp.float32)], axis=1)
    g = _gather_call(xpad, idx3)
    G = g[:, :C].reshape(B, K, N, C)
    return _edge_call(x, G, layers)


def kernel(feat, params):
    B, N, C0 = feat.shape
    bn = params['bn']
    mean = bn['mean'].reshape(1, -1)
    s = jnp.sqrt(bn['var'] + 1e-5).reshape(1, -1)
    gamma = bn['gamma'].reshape(1, -1)
    beta = bn['beta'].reshape(1, -1)
    x0 = _pre_call(feat.reshape(B * N, C0), mean, s, gamma, beta,
                   params['pre']['W'], params['pre']['b'].reshape(1, -1))
    x0 = x0.reshape(B, N, -1)
    x1 = _edge_layer(x0, params['ec'][0])
    f1 = _edge_layer(x1, params['ec'][1])
    f2 = _edge_layer(f1, params['ec'][2])
    f3 = _edge_layer(f2, params['ec'][3])
    return _final_call(f1, f2, f3, params['mlp2'])


# speedup vs baseline: 9.8745x; 1.0866x over previous
import functools
import jax
import jax.numpy as jnp
from jax import lax
from jax.experimental import pallas as pl
from jax.experimental.pallas import tpu as pltpu
from jax.experimental.pallas import tpu_sc as plsc

K = 10
KPAD = 16


def _pre_body(x_ref, mean_ref, s_ref, gamma_ref, beta_ref, w_ref, b_ref, o_ref):
    x = x_ref[...]
    xn = (x - mean_ref[...]) / s_ref[...] * gamma_ref[...] + beta_ref[...]
    o_ref[...] = jax.nn.relu(jnp.dot(xn, w_ref[...]) + b_ref[...])


def _pre_call(x2d, mean, s, gamma, beta, w, b):
    M, C = x2d.shape
    CO = w.shape[1]
    R = 512
    vec = lambda: pl.BlockSpec((1, C), lambda i: (0, 0))
    return pl.pallas_call(
        _pre_body,
        grid=(M // R,),
        in_specs=[
            pl.BlockSpec((R, C), lambda i: (i, 0)),
            vec(), vec(), vec(), vec(),
            pl.BlockSpec((C, CO), lambda i: (0, 0)),
            pl.BlockSpec((1, CO), lambda i: (0, 0)),
        ],
        out_specs=pl.BlockSpec((R, CO), lambda i: (i, 0)),
        out_shape=jax.ShapeDtypeStruct((M, CO), jnp.float32),
    )(x2d, mean, s, gamma, beta, w, b)


def _knn_body(xr_ref, xc_ref, sqr_ref, sqc_ref, idx_ref, *, n_total):
    b = pl.program_id(0)
    xr = xr_ref[0]
    xc = xc_ref[0]
    sq_r = sqr_ref[0, 0]
    sq_c = sqc_ref[0, 0]
    inner = lax.dot_general(xr, xc, (((1,), (1,)), ((), ())),
                            preferred_element_type=jnp.float32)
    dist = sq_r[:, None] - 2.0 * inner + sq_c[None, :]
    cols = lax.broadcasted_iota(jnp.int32, dist.shape, 1)
    base = b * n_total
    for j in range(K):
        idxj = jnp.argmin(dist, axis=1).astype(jnp.int32)
        idx_ref[0, j, :] = idxj + base
        dist = jnp.where(cols == idxj[:, None], jnp.float32(jnp.inf), dist)


def _knn_call(x, sq):
    B, N, C = x.shape
    R = 256
    return pl.pallas_call(
        functools.partial(_knn_body, n_total=N),
        grid=(B, N // R),
        in_specs=[
            pl.BlockSpec((1, R, C), lambda b, r: (b, r, 0)),
            pl.BlockSpec((1, N, C), lambda b, r: (b, 0, 0)),
            pl.BlockSpec((1, 1, R), lambda b, r: (b, 0, r)),
            pl.BlockSpec((1, 1, N), lambda b, r: (b, 0, 0)),
        ],
        out_specs=pl.BlockSpec((1, KPAD, R), lambda b, r: (b, 0, r)),
        out_shape=jax.ShapeDtypeStruct((B, KPAD, N), jnp.int32),
    )(x, x, sq, sq)


def _gather_call(table, idx3):
    V, C = table.shape
    NW, NCHUNK, CH = idx3.shape
    E = NW * NCHUNK * CH
    info = plsc.get_sparse_core_info()
    assert NW == info.num_cores * info.num_subcores

    mesh = plsc.VectorSubcoreMesh(core_axis_name="c", subcore_axis_name="s")

    @functools.partial(
        pl.kernel,
        mesh=mesh,
        out_type=jax.ShapeDtypeStruct((E, C), jnp.float32),
        scratch_types=[
            pltpu.VMEM_SHARED((V, C), jnp.float32),
            pltpu.VMEM((NCHUNK, CH), jnp.int32),
            pltpu.VMEM((CH, C), jnp.float32),
            pltpu.SemaphoreType.DMA,
        ],
    )
    def gk(table_hbm, idx_hbm, out_hbm, table_sp, idx_v, rows_v, sem):
        sid = lax.axis_index("s")
        cid = lax.axis_index("c")
        wid = sid * info.num_cores + cid

        @pl.when(sid == 0)
        def _():
            pltpu.sync_copy(table_hbm, table_sp)

        pltpu.sync_copy(idx_hbm.at[wid], idx_v)
        plsc.subcore_barrier()
        base = wid * (NCHUNK * CH)

        def body(ci, carry):
            pltpu.async_copy(table_sp.at[idx_v.at[ci]], rows_v, sem).wait()
            pltpu.sync_copy(rows_v, out_hbm.at[pl.ds(base + ci * CH, CH)])
            return carry

        lax.fori_loop(0, NCHUNK, body, 0)

    return gk(table, idx3)


def _edge_body(x_ref, g_ref, w1_ref, b1_ref, w2_ref, b2_ref, w3_ref, b3_ref,
               o_ref):
    xc = x_ref[0]
    acc = None
    for j in range(K):
        nb = g_ref[0, j]
        h = jnp.concatenate([xc, nb - xc], axis=-1)
        h = jax.nn.relu(jnp.dot(h, w1_ref[...]) + b1_ref[...])
        h = jax.nn.relu(jnp.dot(h, w2_ref[...]) + b2_ref[...])
        h = jax.nn.relu(jnp.dot(h, w3_ref[...]) + b3_ref[...])
        acc = h if acc is None else jnp.maximum(acc, h)
    o_ref[0] = acc


def _edge_call(x, G, layers):
    B, N, C = x.shape
    R = 512
    CO = layers[2]['W'].shape[1]
    w1, w2, w3 = layers[0]['W'], layers[1]['W'], layers[2]['W']
    b1 = layers[0]['b'].reshape(1, -1)
    b2 = layers[1]['b'].reshape(1, -1)
    b3 = layers[2]['b'].reshape(1, -1)
    wspec = lambda w: pl.BlockSpec(w.shape, lambda b, r: (0, 0))
    return pl.pallas_call(
        _edge_body,
        grid=(B, N // R),
        in_specs=[
            pl.BlockSpec((1, R, C), lambda b, r: (b, r, 0)),
            pl.BlockSpec((1, K, R, C), lambda b, r: (b, 0, r, 0)),
            wspec(w1), wspec(b1), wspec(w2), wspec(b2), wspec(w3), wspec(b3),
        ],
        out_specs=pl.BlockSpec((1, R, CO), lambda b, r: (b, r, 0)),
        out_shape=jax.ShapeDtypeStruct((B, N, CO), jnp.float32),
    )(x, G, w1, b1, w2, b2, w3, b3)


def _final_body(f1_ref, f2_ref, f3_ref, w1_ref, b1_ref, w2_ref, b2_ref, o_ref):
    h = jnp.concatenate([f1_ref[0], f2_ref[0], f3_ref[0]], axis=-1)
    h = jax.nn.relu(jnp.dot(h, w1_ref[...]) + b1_ref[...])
    o_ref[0] = jax.nn.relu(jnp.dot(h, w2_ref[...]) + b2_ref[...])


def _final_call(f1, f2, f3, mlp2):
    B, N, C = f1.shape
    R = 512
    w1, w2 = mlp2[0]['W'], mlp2[1]['W']
    b1 = mlp2[0]['b'].reshape(1, -1)
    b2 = mlp2[1]['b'].reshape(1, -1)
    CO = w2.shape[1]
    fspec = pl.BlockSpec((1, R, C), lambda b, r: (b, r, 0))
    wspec = lambda w: pl.BlockSpec(w.shape, lambda b, r: (0, 0))
    return pl.pallas_call(
        _final_body,
        grid=(B, N // R),
        in_specs=[fspec, fspec, fspec,
                  wspec(w1), wspec(b1), wspec(w2), wspec(b2)],
        out_specs=pl.BlockSpec((1, R, CO), lambda b, r: (b, r, 0)),
        out_shape=jax.ShapeDtypeStruct((B, N, CO), jnp.float32),
    )(f1, f2, f3, w1, b1, w2, b2)


def _edge_layer(x, layers):
    B, N, C = x.shape
    sq = jnp.sum(x * x, axis=-1).reshape(B, 1, N)
    idx = _knn_call(x, sq)
    idx3 = idx[:, :K, :].reshape(32, -1, 128)
    xpad = jnp.concatenate(
        [x.reshape(B * N, C), jnp.zeros((B * N, 128 - C), jnp.float32)], axis=1)
    g = _gather_call(xpad, idx3)
    G = g[:, :C].reshape(B, K, N, C)
    return _edge_call(x, G, layers)


def kernel(feat, params):
    B, N, C0 = feat.shape
    bn = params['bn']
    mean = bn['mean'].reshape(1, -1)
    s = jnp.sqrt(bn['var'] + 1e-5).reshape(1, -1)
    gamma = bn['gamma'].reshape(1, -1)
    beta = bn['beta'].reshape(1, -1)
    x0 = _pre_call(feat.reshape(B * N, C0), mean, s, gamma, beta,
                   params['pre']['W'], params['pre']['b'].reshape(1, -1))
    x0 = x0.reshape(B, N, -1)
    x1 = _edge_layer(x0, params['ec'][0])
    f1 = _edge_layer(x1, params['ec'][1])
    f2 = _edge_layer(f1, params['ec'][2])
    f3 = _edge_layer(f2, params['ec'][3])
    return _final_call(f1, f2, f3, params['mlp2'])


# speedup vs baseline: 10.0860x; 1.0214x over previous
import functools
import jax
import jax.numpy as jnp
from jax import lax
from jax.experimental import pallas as pl
from jax.experimental.pallas import tpu as pltpu
from jax.experimental.pallas import tpu_sc as plsc

K = 10
KPAD = 16


def _pre_body(x_ref, mean_ref, s_ref, gamma_ref, beta_ref, w_ref, b_ref, o_ref):
    x = x_ref[...]
    xn = (x - mean_ref[...]) / s_ref[...] * gamma_ref[...] + beta_ref[...]
    o_ref[...] = jax.nn.relu(jnp.dot(xn, w_ref[...]) + b_ref[...])


def _pre_call(x2d, mean, s, gamma, beta, w, b):
    M, C = x2d.shape
    CO = w.shape[1]
    R = 512
    vec = lambda: pl.BlockSpec((1, C), lambda i: (0, 0))
    return pl.pallas_call(
        _pre_body,
        grid=(M // R,),
        in_specs=[
            pl.BlockSpec((R, C), lambda i: (i, 0)),
            vec(), vec(), vec(), vec(),
            pl.BlockSpec((C, CO), lambda i: (0, 0)),
            pl.BlockSpec((1, CO), lambda i: (0, 0)),
        ],
        out_specs=pl.BlockSpec((R, CO), lambda i: (i, 0)),
        out_shape=jax.ShapeDtypeStruct((M, CO), jnp.float32),
    )(x2d, mean, s, gamma, beta, w, b)


def _knn_body(xr_ref, xc_ref, sqr_ref, sqc_ref, idx_ref, *, n_total):
    b = pl.program_id(0)
    xr = xr_ref[0]
    xc = xc_ref[0]
    sq_r = sqr_ref[0, 0]
    sq_c = sqc_ref[0, 0]
    inner = lax.dot_general(xr, xc, (((1,), (1,)), ((), ())),
                            preferred_element_type=jnp.float32)
    dist = sq_r[:, None] - 2.0 * inner + sq_c[None, :]
    cols = lax.broadcasted_iota(jnp.int32, dist.shape, 1)
    base = b * n_total
    for j in range(K):
        idxj = jnp.argmin(dist, axis=1).astype(jnp.int32)
        idx_ref[0, j, :] = idxj + base
        dist = jnp.where(cols == idxj[:, None], jnp.float32(jnp.inf), dist)


def _knn_call(x, sq):
    B, N, C = x.shape
    R = 256
    return pl.pallas_call(
        functools.partial(_knn_body, n_total=N),
        grid=(B, N // R),
        in_specs=[
            pl.BlockSpec((1, R, C), lambda b, r: (b, r, 0)),
            pl.BlockSpec((1, N, C), lambda b, r: (b, 0, 0)),
            pl.BlockSpec((1, 1, R), lambda b, r: (b, 0, r)),
            pl.BlockSpec((1, 1, N), lambda b, r: (b, 0, 0)),
        ],
        out_specs=pl.BlockSpec((1, KPAD, R), lambda b, r: (b, 0, r)),
        out_shape=jax.ShapeDtypeStruct((B, KPAD, N), jnp.int32),
    )(x, x, sq, sq)


def _gather_call(table, idx3):
    V, C = table.shape
    NW, NCHUNK, CH = idx3.shape
    E = NW * NCHUNK * CH
    info = plsc.get_sparse_core_info()
    assert NW == info.num_cores * info.num_subcores

    mesh = plsc.VectorSubcoreMesh(core_axis_name="c", subcore_axis_name="s")

    @functools.partial(
        pl.kernel,
        mesh=mesh,
        out_type=jax.ShapeDtypeStruct((E, C), jnp.float32),
        scratch_types=[
            pltpu.VMEM_SHARED((V, C), jnp.float32),
            pltpu.VMEM((NCHUNK, CH), jnp.int32),
            pltpu.VMEM((CH, C), jnp.float32),
            pltpu.VMEM((CH, C), jnp.float32),
            pltpu.SemaphoreType.DMA,
            pltpu.SemaphoreType.DMA,
        ],
    )
    def gk(table_hbm, idx_hbm, out_hbm, table_sp, idx_v, rows_a, rows_b, sem_a,
           sem_b):
        sid = lax.axis_index("s")
        cid = lax.axis_index("c")
        wid = sid * info.num_cores + cid

        @pl.when(sid == 0)
        def _():
            pltpu.sync_copy(table_hbm, table_sp)

        pltpu.sync_copy(idx_hbm.at[wid], idx_v)
        plsc.subcore_barrier()
        base = wid * (NCHUNK * CH)

        pltpu.async_copy(table_sp.at[idx_v.at[0]], rows_a, sem_a)

        def body(p, carry):
            ci = 2 * p
            pltpu.make_async_copy(table_sp.at[idx_v.at[ci]], rows_a,
                                  sem_a).wait()
            pltpu.async_copy(table_sp.at[idx_v.at[ci + 1]], rows_b, sem_b)
            pltpu.sync_copy(rows_a, out_hbm.at[pl.ds(base + ci * CH, CH)])
            pltpu.make_async_copy(table_sp.at[idx_v.at[ci + 1]], rows_b,
                                  sem_b).wait()

            @pl.when(ci + 2 < NCHUNK)
            def _():
                pltpu.async_copy(table_sp.at[idx_v.at[ci + 2]], rows_a, sem_a)

            pltpu.sync_copy(rows_b, out_hbm.at[pl.ds(base + (ci + 1) * CH, CH)])
            return carry

        lax.fori_loop(0, NCHUNK // 2, body, 0)

    return gk(table, idx3)


def _edge_body(x_ref, g_ref, w1_ref, b1_ref, w2_ref, b2_ref, w3_ref, b3_ref,
               o_ref):
    xc = x_ref[0]
    acc = None
    for j in range(K):
        nb = g_ref[0, j]
        h = jnp.concatenate([xc, nb - xc], axis=-1)
        h = jax.nn.relu(jnp.dot(h, w1_ref[...]) + b1_ref[...])
        h = jax.nn.relu(jnp.dot(h, w2_ref[...]) + b2_ref[...])
        h = jax.nn.relu(jnp.dot(h, w3_ref[...]) + b3_ref[...])
        acc = h if acc is None else jnp.maximum(acc, h)
    o_ref[0] = acc


def _edge_call(x, G, layers):
    B, N, C = x.shape
    R = 512
    CO = layers[2]['W'].shape[1]
    w1, w2, w3 = layers[0]['W'], layers[1]['W'], layers[2]['W']
    b1 = layers[0]['b'].reshape(1, -1)
    b2 = layers[1]['b'].reshape(1, -1)
    b3 = layers[2]['b'].reshape(1, -1)
    wspec = lambda w: pl.BlockSpec(w.shape, lambda b, r: (0, 0))
    return pl.pallas_call(
        _edge_body,
        grid=(B, N // R),
        in_specs=[
            pl.BlockSpec((1, R, C), lambda b, r: (b, r, 0)),
            pl.BlockSpec((1, K, R, C), lambda b, r: (b, 0, r, 0)),
            wspec(w1), wspec(b1), wspec(w2), wspec(b2), wspec(w3), wspec(b3),
        ],
        out_specs=pl.BlockSpec((1, R, CO), lambda b, r: (b, r, 0)),
        out_shape=jax.ShapeDtypeStruct((B, N, CO), jnp.float32),
    )(x, G, w1, b1, w2, b2, w3, b3)


def _final_body(f1_ref, f2_ref, f3_ref, w1_ref, b1_ref, w2_ref, b2_ref, o_ref):
    h = jnp.concatenate([f1_ref[0], f2_ref[0], f3_ref[0]], axis=-1)
    h = jax.nn.relu(jnp.dot(h, w1_ref[...]) + b1_ref[...])
    o_ref[0] = jax.nn.relu(jnp.dot(h, w2_ref[...]) + b2_ref[...])


def _final_call(f1, f2, f3, mlp2):
    B, N, C = f1.shape
    R = 512
    w1, w2 = mlp2[0]['W'], mlp2[1]['W']
    b1 = mlp2[0]['b'].reshape(1, -1)
    b2 = mlp2[1]['b'].reshape(1, -1)
    CO = w2.shape[1]
    fspec = pl.BlockSpec((1, R, C), lambda b, r: (b, r, 0))
    wspec = lambda w: pl.BlockSpec(w.shape, lambda b, r: (0, 0))
    return pl.pallas_call(
        _final_body,
        grid=(B, N // R),
        in_specs=[fspec, fspec, fspec,
                  wspec(w1), wspec(b1), wspec(w2), wspec(b2)],
        out_specs=pl.BlockSpec((1, R, CO), lambda b, r: (b, r, 0)),
        out_shape=jax.ShapeDtypeStruct((B, N, CO), jnp.float32),
    )(f1, f2, f3, w1, b1, w2, b2)


def _edge_layer(x, layers):
    B, N, C = x.shape
    sq = jnp.sum(x * x, axis=-1).reshape(B, 1, N)
    idx = _knn_call(x, sq)
    idx3 = idx[:, :K, :].reshape(32, -1, 128)
    xpad = jnp.concatenate(
        [x.reshape(B * N, C), jnp.zeros((B * N, 128 - C), jnp.float32)], axis=1)
    g = _gather_call(xpad, idx3)
    G = g[:, :C].reshape(B, K, N, C)
    return _edge_call(x, G, layers)


def kernel(feat, params):
    B, N, C0 = feat.shape
    bn = params['bn']
    mean = bn['mean'].reshape(1, -1)
    s = jnp.sqrt(bn['var'] + 1e-5).reshape(1, -1)
    gamma = bn['gamma'].reshape(1, -1)
    beta = bn['beta'].reshape(1, -1)
    x0 = _pre_call(feat.reshape(B * N, C0), mean, s, gamma, beta,
                   params['pre']['W'], params['pre']['b'].reshape(1, -1))
    x0 = x0.reshape(B, N, -1)
    x1 = _edge_layer(x0, params['ec'][0])
    f1 = _edge_layer(x1, params['ec'][1])
    f2 = _edge_layer(f1, params['ec'][2])
    f3 = _edge_layer(f2, params['ec'][3])
    return _final_call(f1, f2, f3, params['mlp2'])


# speedup vs baseline: 10.1403x; 1.0054x over previous
import functools
import jax
import jax.numpy as jnp
from jax import lax
from jax.experimental import pallas as pl
from jax.experimental.pallas import tpu as pltpu
from jax.experimental.pallas import tpu_sc as plsc

K = 10
KPAD = 16


def _pre_body(x_ref, mean_ref, s_ref, gamma_ref, beta_ref, w_ref, b_ref, o_ref):
    x = x_ref[...]
    xn = (x - mean_ref[...]) / s_ref[...] * gamma_ref[...] + beta_ref[...]
    h = jax.nn.relu(jnp.dot(xn, w_ref[...]) + b_ref[...])
    o_ref[...] = jnp.concatenate(
        [h, jnp.zeros((h.shape[0], 128 - h.shape[1]), jnp.float32)], axis=-1)


def _pre_call(x2d, mean, s, gamma, beta, w, b):
    M, C = x2d.shape
    CO = w.shape[1]
    R = 512
    vec = lambda: pl.BlockSpec((1, C), lambda i: (0, 0))
    return pl.pallas_call(
        _pre_body,
        grid=(M // R,),
        in_specs=[
            pl.BlockSpec((R, C), lambda i: (i, 0)),
            vec(), vec(), vec(), vec(),
            pl.BlockSpec((C, CO), lambda i: (0, 0)),
            pl.BlockSpec((1, CO), lambda i: (0, 0)),
        ],
        out_specs=pl.BlockSpec((R, 128), lambda i: (i, 0)),
        out_shape=jax.ShapeDtypeStruct((M, 128), jnp.float32),
    )(x2d, mean, s, gamma, beta, w, b)


def _knn_body(xr_ref, xc_ref, sqr_ref, sqc_ref, idx_ref, *, n_total, c):
    b = pl.program_id(0)
    xr = xr_ref[0][:, :c]
    xc = xc_ref[0][:, :c]
    sq_r = sqr_ref[0, 0]
    sq_c = sqc_ref[0, 0]
    inner = lax.dot_general(xr, xc, (((1,), (1,)), ((), ())),
                            preferred_element_type=jnp.float32)
    dist = sq_r[:, None] - 2.0 * inner + sq_c[None, :]
    cols = lax.broadcasted_iota(jnp.int32, dist.shape, 1)
    base = b * n_total
    for j in range(K):
        idxj = jnp.argmin(dist, axis=1).astype(jnp.int32)
        idx_ref[0, j, :] = idxj + base
        dist = jnp.where(cols == idxj[:, None], jnp.float32(jnp.inf), dist)


def _knn_call(x128, sq, c):
    B, N, _ = x128.shape
    R = 256
    return pl.pallas_call(
        functools.partial(_knn_body, n_total=N, c=c),
        grid=(B, N // R),
        in_specs=[
            pl.BlockSpec((1, R, 128), lambda b, r: (b, r, 0)),
            pl.BlockSpec((1, N, 128), lambda b, r: (b, 0, 0)),
            pl.BlockSpec((1, 1, R), lambda b, r: (b, 0, r)),
            pl.BlockSpec((1, 1, N), lambda b, r: (b, 0, 0)),
        ],
        out_specs=pl.BlockSpec((1, KPAD, R), lambda b, r: (b, 0, r)),
        out_shape=jax.ShapeDtypeStruct((B, KPAD, N), jnp.int32),
    )(x128, x128, sq, sq)


def _gather_call(table, idx3):
    V, C = table.shape
    NW, NCHUNK, CH = idx3.shape
    E = NW * NCHUNK * CH
    info = plsc.get_sparse_core_info()
    assert NW == info.num_cores * info.num_subcores

    mesh = plsc.VectorSubcoreMesh(core_axis_name="c", subcore_axis_name="s")

    @functools.partial(
        pl.kernel,
        mesh=mesh,
        out_type=jax.ShapeDtypeStruct((E, C), jnp.float32),
        scratch_types=[
            pltpu.VMEM_SHARED((V, C), jnp.float32),
            pltpu.VMEM((NCHUNK, CH), jnp.int32),
            pltpu.VMEM((CH, C), jnp.float32),
            pltpu.VMEM((CH, C), jnp.float32),
            pltpu.SemaphoreType.DMA,
            pltpu.SemaphoreType.DMA,
        ],
    )
    def gk(table_hbm, idx_hbm, out_hbm, table_sp, idx_v, rows_a, rows_b, sem_a,
           sem_b):
        sid = lax.axis_index("s")
        cid = lax.axis_index("c")
        wid = sid * info.num_cores + cid

        @pl.when(sid == 0)
        def _():
            pltpu.sync_copy(table_hbm, table_sp)

        pltpu.sync_copy(idx_hbm.at[wid], idx_v)
        plsc.subcore_barrier()
        base = wid * (NCHUNK * CH)

        pltpu.async_copy(table_sp.at[idx_v.at[0]], rows_a, sem_a)

        def body(p, carry):
            ci = 2 * p
            pltpu.make_async_copy(table_sp.at[idx_v.at[ci]], rows_a,
                                  sem_a).wait()
            pltpu.async_copy(table_sp.at[idx_v.at[ci + 1]], rows_b, sem_b)
            pltpu.sync_copy(rows_a, out_hbm.at[pl.ds(base + ci * CH, CH)])
            pltpu.make_async_copy(table_sp.at[idx_v.at[ci + 1]], rows_b,
                                  sem_b).wait()

            @pl.when(ci + 2 < NCHUNK)
            def _():
                pltpu.async_copy(table_sp.at[idx_v.at[ci + 2]], rows_a, sem_a)

            pltpu.sync_copy(rows_b, out_hbm.at[pl.ds(base + (ci + 1) * CH, CH)])
            return carry

        lax.fori_loop(0, NCHUNK // 2, body, 0)

    return gk(table, idx3)


def _edge_body(x_ref, g_ref, w1_ref, b1_ref, w2_ref, b2_ref, w3_ref, b3_ref,
               o_ref, *, c):
    xc = x_ref[0][:, :c]
    acc = None
    for j in range(K):
        nb = g_ref[0, j][:, :c]
        h = jnp.concatenate([xc, nb - xc], axis=-1)
        h = jax.nn.relu(jnp.dot(h, w1_ref[...]) + b1_ref[...])
        h = jax.nn.relu(jnp.dot(h, w2_ref[...]) + b2_ref[...])
        h = jax.nn.relu(jnp.dot(h, w3_ref[...]) + b3_ref[...])
        acc = h if acc is None else jnp.maximum(acc, h)
    co = acc.shape[-1]
    o_ref[0] = jnp.concatenate(
        [acc, jnp.zeros((acc.shape[0], 128 - co), jnp.float32)], axis=-1)


def _edge_call(x128, G4, layers, c):
    B, N, _ = x128.shape
    R = 512
    CO = layers[2]['W'].shape[1]
    w1, w2, w3 = layers[0]['W'], layers[1]['W'], layers[2]['W']
    b1 = layers[0]['b'].reshape(1, -1)
    b2 = layers[1]['b'].reshape(1, -1)
    b3 = layers[2]['b'].reshape(1, -1)
    wspec = lambda w: pl.BlockSpec(w.shape, lambda b, r: (0, 0))
    return pl.pallas_call(
        functools.partial(_edge_body, c=c),
        grid=(B, N // R),
        in_specs=[
            pl.BlockSpec((1, R, 128), lambda b, r: (b, r, 0)),
            pl.BlockSpec((1, K, R, 128), lambda b, r: (b, 0, r, 0)),
            wspec(w1), wspec(b1), wspec(w2), wspec(b2), wspec(w3), wspec(b3),
        ],
        out_specs=pl.BlockSpec((1, R, 128), lambda b, r: (b, r, 0)),
        out_shape=jax.ShapeDtypeStruct((B, N, 128), jnp.float32),
    )(x128, G4, w1, b1, w2, b2, w3, b3)


def _final_body(f1_ref, f2_ref, f3_ref, w1_ref, b1_ref, w2_ref, b2_ref, o_ref,
                *, c):
    h = jnp.concatenate(
        [f1_ref[0][:, :c], f2_ref[0][:, :c], f3_ref[0][:, :c]], axis=-1)
    h = jax.nn.relu(jnp.dot(h, w1_ref[...]) + b1_ref[...])
    o_ref[0] = jax.nn.relu(jnp.dot(h, w2_ref[...]) + b2_ref[...])


def _final_call(f1, f2, f3, mlp2, c):
    B, N, _ = f1.shape
    R = 512
    w1, w2 = mlp2[0]['W'], mlp2[1]['W']
    b1 = mlp2[0]['b'].reshape(1, -1)
    b2 = mlp2[1]['b'].reshape(1, -1)
    CO = w2.shape[1]
    fspec = pl.BlockSpec((1, R, 128), lambda b, r: (b, r, 0))
    wspec = lambda w: pl.BlockSpec(w.shape, lambda b, r: (0, 0))
    return pl.pallas_call(
        functools.partial(_final_body, c=c),
        grid=(B, N // R),
        in_specs=[fspec, fspec, fspec,
                  wspec(w1), wspec(b1), wspec(w2), wspec(b2)],
        out_specs=pl.BlockSpec((1, R, CO), lambda b, r: (b, r, 0)),
        out_shape=jax.ShapeDtypeStruct((B, N, CO), jnp.float32),
    )(f1, f2, f3, w1, b1, w2, b2)


def _edge_layer(x128, layers, c):
    B, N, _ = x128.shape
    xs = x128[:, :, :c]
    sq = jnp.sum(xs * xs, axis=-1).reshape(B, 1, N)
    idx = _knn_call(x128, sq, c)
    idx3 = idx[:, :K, :].reshape(32, -1, 128)
    g = _gather_call(x128.reshape(B * N, 128), idx3)
    G4 = g.reshape(B, K, N, 128)
    return _edge_call(x128, G4, layers, c)


def kernel(feat, params):
    B, N, C0 = feat.shape
    CH = 64
    bn = params['bn']
    mean = bn['mean'].reshape(1, -1)
    s = jnp.sqrt(bn['var'] + 1e-5).reshape(1, -1)
    gamma = bn['gamma'].reshape(1, -1)
    beta = bn['beta'].reshape(1, -1)
    x0 = _pre_call(feat.reshape(B * N, C0), mean, s, gamma, beta,
                   params['pre']['W'], params['pre']['b'].reshape(1, -1))
    x0 = x0.reshape(B, N, 128)
    x1 = _edge_layer(x0, params['ec'][0], CH)
    f1 = _edge_layer(x1, params['ec'][1], CH)
    f2 = _edge_layer(f1, params['ec'][2], CH)
    f3 = _edge_layer(f2, params['ec'][3], CH)
    return _final_call(f1, f2, f3, params['mlp2'], CH)
